# Initial kernel scaffold; baseline (speedup 1.0000x reference)
import functools

import jax
import jax.numpy as jnp
from jax import lax
from jax.experimental import pallas as pl
from jax.experimental.pallas import tpu as pltpu
from jax.experimental.pallas import tpu_sc as plsc

VOCAB = 1_000_000
DIM = 64
BATCH = 4096
CTX = 20
NEG = 10

NPOS = CTX
NNEG = CTX * NEG
PAD = 4
ROWS_B = NPOS + NNEG + PAD
LANES = 16
GROUPS_B = ROWS_B // LANES

NWORKERS = 32
B_PER_W = BATCH // NWORKERS
CHUNK_B = 2
CHUNK_ROWS = CHUNK_B * ROWS_B
N_CHUNKS = B_PER_W // CHUNK_B
GATHERS = 4
GLEN = CHUNK_ROWS // GATHERS
TOTAL_ROWS = BATCH * ROWS_B
TC_COLS = 128
TC_ROWS = TOTAL_ROWS // TC_COLS
TC_GRID = 8
TC_BLK = TC_ROWS // TC_GRID


def _sc_body(idx_hbm, inword_hbm, win_hbm, wout_hbm, out_hbm,
             inidx_v, inrows_v, cidx_v, rows_v, sc_v, sem):
    nc = 2
    wid = lax.axis_index("s") * nc + lax.axis_index("c")

    pltpu.sync_copy(inword_hbm.at[pl.ds(wid * B_PER_W, B_PER_W)], inidx_v)
    pltpu.async_copy(win_hbm.at[inidx_v], inrows_v, sem).wait()

    iota = lax.iota(jnp.int32, LANES)

    def chunk_body(chunk, carry):
        gchunk = wid * N_CHUNKS + chunk
        pltpu.sync_copy(idx_hbm.at[gchunk], cidx_v)
        copies = [
            pltpu.async_copy(
                wout_hbm.at[cidx_v.at[j]],
                rows_v.at[pl.ds(j * GLEN, GLEN)],
                sem,
            )
            for j in range(GATHERS)
        ]
        for cp in copies:
            cp.wait()

        def group_body(g, gc):
            b_idx = chunk * CHUNK_B + g // GROUPS_B
            row_ids = g * LANES + iota
            acc = jnp.zeros((LANES,), jnp.float32)
            for d in range(DIM):
                col = plsc.load_gather(
                    rows_v, [row_ids, jnp.full((LANES,), d, jnp.int32)])
                acc = acc + col * inrows_v[b_idx, d]
            sc_v[pl.ds(g * LANES, LANES)] = acc
            return gc

        lax.fori_loop(0, CHUNK_B * GROUPS_B, group_body, 0)
        pltpu.sync_copy(sc_v, out_hbm.at[pl.ds(gchunk * CHUNK_ROWS, CHUNK_ROWS)])
        return carry

    lax.fori_loop(0, N_CHUNKS, chunk_body, 0)


def _sc_scores(idx3, inword, W_in, W_out):
    mesh = plsc.VectorSubcoreMesh(core_axis_name="c", subcore_axis_name="s")
    k = functools.partial(
        pl.kernel,
        mesh=mesh,
        out_type=jax.ShapeDtypeStruct((TOTAL_ROWS,), jnp.float32),
        scratch_types=[
            pltpu.VMEM((B_PER_W,), jnp.int32),
            pltpu.VMEM((B_PER_W, DIM), jnp.float32),
            pltpu.VMEM((GATHERS, GLEN), jnp.int32),
            pltpu.VMEM((CHUNK_ROWS, DIM), jnp.float32),
            pltpu.VMEM((CHUNK_ROWS,), jnp.float32),
            pltpu.SemaphoreType.DMA,
        ],
    )(_sc_body)
    return k(idx3, inword, W_in, W_out)


def _tc_loss_body(s_ref, o_ref):
    pid = pl.program_id(0)
    x = s_ref[...]
    r = lax.broadcasted_iota(jnp.int32, (TC_BLK, TC_COLS), 0)
    c = lax.broadcasted_iota(jnp.int32, (TC_BLK, TC_COLS), 1)
    flat = (pid * TC_BLK + r) * TC_COLS + c
    j = flat % ROWS_B
    z = jnp.where(j < NPOS, x, -x)
    ls = jnp.minimum(z, 0.0) - jnp.log(1.0 + jnp.exp(-jnp.abs(z)))
    val = jnp.where(j < NPOS + NNEG, ls, 0.0)

    @pl.when(pid == 0)
    def _init():
        o_ref[0, 0] = 0.0

    o_ref[0, 0] += jnp.sum(val)

    @pl.when(pid == TC_GRID - 1)
    def _fini():
        o_ref[0, 0] = o_ref[0, 0] * (-1.0 / (BATCH * CTX))


def kernel(inword, outword, negword, W_in, W_out):
    pad = jnp.zeros((BATCH, PAD), jnp.int32)
    idx3 = (
        jnp.concatenate([outword, negword, pad], axis=1)
        .reshape(-1, GATHERS, GLEN)
    )
    scores = _sc_scores(idx3, inword, W_in, W_out)
    loss2d = pl.pallas_call(
        _tc_loss_body,
        grid=(TC_GRID,),
        in_specs=[pl.BlockSpec((TC_BLK, TC_COLS), lambda i: (i, 0))],
        out_specs=pl.BlockSpec(
            (1, 1), lambda i: (0, 0), memory_space=pltpu.SMEM),
        out_shape=jax.ShapeDtypeStruct((1, 1), jnp.float32),
    )(scores.reshape(TC_ROWS, TC_COLS))
    return loss2d[0, 0]


# speedup vs baseline: 1.1058x; 1.1058x over previous
import functools

import jax
import jax.numpy as jnp
from jax import lax
from jax.experimental import pallas as pl
from jax.experimental.pallas import tpu as pltpu
from jax.experimental.pallas import tpu_sc as plsc

VOCAB = 1_000_000
DIM = 64
BATCH = 4096
CTX = 20
NEG = 10

NPOS = CTX
NNEG = CTX * NEG
PAD = 4
ROWS_B = NPOS + NNEG + PAD
LANES = 16
GROUPS_B = ROWS_B // LANES

NWORKERS = 32
B_PER_W = BATCH // NWORKERS
CHUNK_B = 2
CHUNK_ROWS = CHUNK_B * ROWS_B
N_CHUNKS = B_PER_W // CHUNK_B
GATHERS = 4
GLEN = CHUNK_ROWS // GATHERS
TOTAL_ROWS = BATCH * ROWS_B
TC_COLS = 128
TC_ROWS = TOTAL_ROWS // TC_COLS
TC_GRID = 8
TC_BLK = TC_ROWS // TC_GRID


def _sc_body(idx_hbm, inword_hbm, win_hbm, wout_hbm, out_hbm,
             inidx_v, inrows_v, cidx_v, rows_v, sc_v, sem):
    nc = 2
    wid = lax.axis_index("s") * nc + lax.axis_index("c")

    pltpu.sync_copy(inword_hbm.at[pl.ds(wid * B_PER_W, B_PER_W)], inidx_v)
    pltpu.async_copy(win_hbm.at[inidx_v], inrows_v, sem).wait()

    iota = lax.iota(jnp.int32, LANES)

    def chunk_body(chunk, carry):
        gchunk = wid * N_CHUNKS + chunk
        pltpu.sync_copy(idx_hbm.at[gchunk], cidx_v)
        copies = [
            pltpu.async_copy(
                wout_hbm.at[cidx_v.at[j]],
                rows_v.at[pl.ds(j * GLEN, GLEN)],
                sem,
            )
            for j in range(GATHERS)
        ]
        for cp in copies:
            cp.wait()

        def group_body(g, gc):
            b_idx = chunk * CHUNK_B + g // GROUPS_B
            wv = [inrows_v[b_idx, pl.ds(q * LANES, LANES)]
                  for q in range(DIM // LANES)]
            acc = jnp.zeros((LANES,), jnp.float32)
            row_ids = g * LANES + iota
            for d in range(DIM):
                col = plsc.load_gather(
                    rows_v, [row_ids, jnp.full((LANES,), d, jnp.int32)])
                acc = acc + col * wv[d // LANES][d % LANES]
            sc_v[pl.ds(g * LANES, LANES)] = acc
            return gc

        lax.fori_loop(0, CHUNK_B * GROUPS_B, group_body, 0)
        pltpu.sync_copy(sc_v, out_hbm.at[pl.ds(gchunk * CHUNK_ROWS, CHUNK_ROWS)])
        return carry

    lax.fori_loop(0, N_CHUNKS, chunk_body, 0)


def _sc_scores(idx3, inword, W_in, W_out):
    mesh = plsc.VectorSubcoreMesh(core_axis_name="c", subcore_axis_name="s")
    k = functools.partial(
        pl.kernel,
        mesh=mesh,
        out_type=jax.ShapeDtypeStruct((TOTAL_ROWS,), jnp.float32),
        compiler_params=pltpu.CompilerParams(
            needs_layout_passes=False, use_tc_tiling_on_sc=False),
        scratch_types=[
            pltpu.VMEM((B_PER_W,), jnp.int32),
            pltpu.VMEM((B_PER_W, DIM), jnp.float32),
            pltpu.VMEM((GATHERS, GLEN), jnp.int32),
            pltpu.VMEM((CHUNK_ROWS, DIM), jnp.float32),
            pltpu.VMEM((CHUNK_ROWS,), jnp.float32),
            pltpu.SemaphoreType.DMA,
        ],
    )(_sc_body)
    return k(idx3, inword, W_in, W_out)


def _tc_loss_body(s_ref, o_ref):
    pid = pl.program_id(0)
    x = s_ref[...]
    r = lax.broadcasted_iota(jnp.int32, (TC_BLK, TC_COLS), 0)
    c = lax.broadcasted_iota(jnp.int32, (TC_BLK, TC_COLS), 1)
    flat = (pid * TC_BLK + r) * TC_COLS + c
    j = flat % ROWS_B
    z = jnp.where(j < NPOS, x, -x)
    ls = jnp.minimum(z, 0.0) - jnp.log(1.0 + jnp.exp(-jnp.abs(z)))
    val = jnp.where(j < NPOS + NNEG, ls, 0.0)

    @pl.when(pid == 0)
    def _init():
        o_ref[0, 0] = 0.0

    o_ref[0, 0] += jnp.sum(val)

    @pl.when(pid == TC_GRID - 1)
    def _fini():
        o_ref[0, 0] = o_ref[0, 0] * (-1.0 / (BATCH * CTX))


def kernel(inword, outword, negword, W_in, W_out):
    pad = jnp.zeros((BATCH, PAD), jnp.int32)
    idx3 = (
        jnp.concatenate([outword, negword, pad], axis=1)
        .reshape(-1, GATHERS, GLEN)
    )
    scores = _sc_scores(idx3, inword, W_in, W_out)
    loss2d = pl.pallas_call(
        _tc_loss_body,
        grid=(TC_GRID,),
        in_specs=[pl.BlockSpec((TC_BLK, TC_COLS), lambda i: (i, 0))],
        out_specs=pl.BlockSpec(
            (1, 1), lambda i: (0, 0), memory_space=pltpu.SMEM),
        out_shape=jax.ShapeDtypeStruct((1, 1), jnp.float32),
    )(scores.reshape(TC_ROWS, TC_COLS))
    return loss2d[0, 0]


# speedup vs baseline: 1.1573x; 1.0466x over previous
import functools

import jax
import jax.numpy as jnp
from jax import lax
from jax.experimental import pallas as pl
from jax.experimental.pallas import tpu as pltpu
from jax.experimental.pallas import tpu_sc as plsc

VOCAB = 1_000_000
DIM = 64
BATCH = 4096
CTX = 20
NEG = 10

NPOS = CTX
NNEG = CTX * NEG
PAD = 4
ROWS_B = NPOS + NNEG + PAD
LANES = 16
GROUPS_B = ROWS_B // LANES

NWORKERS = 32
B_PER_W = BATCH // NWORKERS
CHUNK_B = 2
CHUNK_ROWS = CHUNK_B * ROWS_B
N_CHUNKS = B_PER_W // CHUNK_B
GATHERS = 4
GLEN = CHUNK_ROWS // GATHERS
TOTAL_ROWS = BATCH * ROWS_B
TC_COLS = 128
TC_ROWS = TOTAL_ROWS // TC_COLS
TC_GRID = 8
TC_BLK = TC_ROWS // TC_GRID


def _sc_body(idx_hbm, inword_hbm, win_hbm, wout_hbm, out_hbm,
             inidx_v, inrows_v, cidx_v, rows_v, sc_v,
             sem_row0, sem_row1, sem_idx0, sem_idx1, sem_sc0, sem_sc1):
    nc = 2
    wid = lax.axis_index("s") * nc + lax.axis_index("c")
    sem_row = (sem_row0, sem_row1)
    sem_idx = (sem_idx0, sem_idx1)
    sem_sc = (sem_sc0, sem_sc1)

    pltpu.sync_copy(inword_hbm.at[pl.ds(wid * B_PER_W, B_PER_W)], inidx_v)
    pltpu.async_copy(win_hbm.at[inidx_v], inrows_v, sem_row0).wait()

    iota = lax.iota(jnp.int32, LANES)
    base_g = wid * N_CHUNKS

    def fire_rows(par):
        for j in range(GATHERS):
            pltpu.async_copy(
                wout_hbm.at[cidx_v.at[par, j]],
                rows_v.at[par, pl.ds(j * GLEN, GLEN)],
                sem_row[par])

    def drain_rows(par):
        for j in range(GATHERS):
            pltpu.make_async_copy(
                wout_hbm.at[pl.ds(0, GLEN)],
                rows_v.at[par, pl.ds(j * GLEN, GLEN)],
                sem_row[par]).wait()

    pltpu.sync_copy(idx_hbm.at[base_g], cidx_v.at[0])
    fire_rows(0)
    pltpu.async_copy(idx_hbm.at[base_g + 1], cidx_v.at[1], sem_idx1)

    def pair_body(cc, carry):
        for par in (0, 1):
            chunk = cc * 2 + par
            gchunk = base_g + chunk
            npar = 1 - par

            drain_rows(par)

            @pl.when(chunk + 1 < N_CHUNKS)
            def _fire():
                pltpu.make_async_copy(
                    idx_hbm.at[base_g], cidx_v.at[npar],
                    sem_idx[npar]).wait()
                fire_rows(npar)

            @pl.when(chunk + 2 < N_CHUNKS)
            def _pref():
                pltpu.async_copy(
                    idx_hbm.at[gchunk + 2], cidx_v.at[par], sem_idx[par])

            @pl.when(chunk >= 2)
            def _drain_sc():
                pltpu.make_async_copy(
                    out_hbm.at[pl.ds(0, CHUNK_ROWS)], sc_v.at[par],
                    sem_sc[par]).wait()

            def group_body(g, gc):
                b_idx = chunk * CHUNK_B + g // GROUPS_B
                row_ids = g * LANES + iota
                wv = [inrows_v[b_idx, pl.ds(q * LANES, LANES)]
                      for q in range(DIM // LANES)]
                acc = [jnp.zeros((LANES,), jnp.float32) for _ in range(4)]
                for d in range(DIM):
                    col = plsc.load_gather(
                        rows_v.at[par],
                        [row_ids, jnp.full((LANES,), d, jnp.int32)])
                    acc[d % 4] = acc[d % 4] + col * wv[d // LANES][d % LANES]
                sc_v[par, pl.ds(g * LANES, LANES)] = (
                    (acc[0] + acc[1]) + (acc[2] + acc[3]))
                return gc

            lax.fori_loop(0, CHUNK_B * GROUPS_B, group_body, 0)
            pltpu.async_copy(
                sc_v.at[par],
                out_hbm.at[pl.ds(gchunk * CHUNK_ROWS, CHUNK_ROWS)],
                sem_sc[par])
        return carry

    lax.fori_loop(0, N_CHUNKS // 2, pair_body, 0)
    for par in (0, 1):
        pltpu.make_async_copy(
            out_hbm.at[pl.ds(0, CHUNK_ROWS)], sc_v.at[par],
            sem_sc[par]).wait()


def _sc_scores(idx3, inword, W_in, W_out):
    mesh = plsc.VectorSubcoreMesh(core_axis_name="c", subcore_axis_name="s")
    k = functools.partial(
        pl.kernel,
        mesh=mesh,
        out_type=jax.ShapeDtypeStruct((TOTAL_ROWS,), jnp.float32),
        compiler_params=pltpu.CompilerParams(
            needs_layout_passes=False, use_tc_tiling_on_sc=False),
        scratch_types=[
            pltpu.VMEM((B_PER_W,), jnp.int32),
            pltpu.VMEM((B_PER_W, DIM), jnp.float32),
            pltpu.VMEM((2, GATHERS, GLEN), jnp.int32),
            pltpu.VMEM((2, CHUNK_ROWS, DIM), jnp.float32),
            pltpu.VMEM((2, CHUNK_ROWS), jnp.float32),
            pltpu.SemaphoreType.DMA,
            pltpu.SemaphoreType.DMA,
            pltpu.SemaphoreType.DMA,
            pltpu.SemaphoreType.DMA,
            pltpu.SemaphoreType.DMA,
            pltpu.SemaphoreType.DMA,
        ],
    )(_sc_body)
    return k(idx3, inword, W_in, W_out)


def _tc_loss_body(s_ref, o_ref):
    pid = pl.program_id(0)
    x = s_ref[...]
    r = lax.broadcasted_iota(jnp.int32, (TC_BLK, TC_COLS), 0)
    c = lax.broadcasted_iota(jnp.int32, (TC_BLK, TC_COLS), 1)
    flat = (pid * TC_BLK + r) * TC_COLS + c
    j = flat % ROWS_B
    z = jnp.where(j < NPOS, x, -x)
    ls = jnp.minimum(z, 0.0) - jnp.log(1.0 + jnp.exp(-jnp.abs(z)))
    val = jnp.where(j < NPOS + NNEG, ls, 0.0)

    @pl.when(pid == 0)
    def _init():
        o_ref[0, 0] = 0.0

    o_ref[0, 0] += jnp.sum(val)

    @pl.when(pid == TC_GRID - 1)
    def _fini():
        o_ref[0, 0] = o_ref[0, 0] * (-1.0 / (BATCH * CTX))


def kernel(inword, outword, negword, W_in, W_out):
    pad = jnp.zeros((BATCH, PAD), jnp.int32)
    idx3 = (
        jnp.concatenate([outword, negword, pad], axis=1)
        .reshape(-1, GATHERS, GLEN)
    )
    scores = _sc_scores(idx3, inword, W_in, W_out)
    loss2d = pl.pallas_call(
        _tc_loss_body,
        grid=(TC_GRID,),
        in_specs=[pl.BlockSpec((TC_BLK, TC_COLS), lambda i: (i, 0))],
        out_specs=pl.BlockSpec(
            (1, 1), lambda i: (0, 0), memory_space=pltpu.SMEM),
        out_shape=jax.ShapeDtypeStruct((1, 1), jnp.float32),
    )(scores.reshape(TC_ROWS, TC_COLS))
    return loss2d[0, 0]


# speedup vs baseline: 1.4531x; 1.2556x over previous
import functools

import jax
import jax.numpy as jnp
from jax import lax
from jax.experimental import pallas as pl
from jax.experimental.pallas import tpu as pltpu
from jax.experimental.pallas import tpu_sc as plsc

VOCAB = 1_000_000
DIM = 64
BATCH = 4096
CTX = 20
NEG = 10

NPOS = CTX
NPOSP = 24
NNEG = CTX * NEG
ROWS_B = NPOSP + NNEG
LANES = 16
GROUPS_B = ROWS_B // LANES

NWORKERS = 32
B_PER_W = BATCH // NWORKERS
CHUNK_B = 2
CHUNK_ROWS = CHUNK_B * ROWS_B
N_CHUNKS = B_PER_W // CHUNK_B
GATHERS = 4
GLEN = CHUNK_ROWS // GATHERS
TOTAL_ROWS = BATCH * ROWS_B
TC_COLS = 128
TC_ROWS = TOTAL_ROWS // TC_COLS
TC_GRID = 8
TC_BLK = TC_ROWS // TC_GRID


def _sc_body(inword_hbm, owp_hbm, nw_hbm, win_hbm, wout_hbm, out_hbm,
             inidx_v, inrows_v, cidx_v, rows_v, sc_v,
             sem_row0, sem_row1, sem_idx0, sem_idx1, sem_sc0, sem_sc1):
    nc = 2
    wid = lax.axis_index("s") * nc + lax.axis_index("c")
    sem_row = (sem_row0, sem_row1)
    sem_idx = (sem_idx0, sem_idx1)
    sem_sc = (sem_sc0, sem_sc1)

    pltpu.sync_copy(inword_hbm.at[pl.ds(wid * B_PER_W, B_PER_W)], inidx_v)
    pltpu.async_copy(win_hbm.at[inidx_v], inrows_v, sem_row0).wait()

    iota = lax.iota(jnp.int32, LANES)
    base_b = wid * B_PER_W

    def fire_idx(chunk, par):
        b0 = base_b + chunk * CHUNK_B
        for b_loc in range(CHUNK_B):
            off = b_loc * ROWS_B
            pltpu.async_copy(
                owp_hbm.at[b0 + b_loc],
                cidx_v.at[par, pl.ds(off, NPOSP)], sem_idx[par])
            pltpu.async_copy(
                nw_hbm.at[b0 + b_loc],
                cidx_v.at[par, pl.ds(off + NPOSP, NNEG)], sem_idx[par])

    def drain_idx(par):
        for _ in range(CHUNK_B):
            pltpu.make_async_copy(
                owp_hbm.at[0], cidx_v.at[par, pl.ds(0, NPOSP)],
                sem_idx[par]).wait()
            pltpu.make_async_copy(
                nw_hbm.at[0], cidx_v.at[par, pl.ds(NPOSP, NNEG)],
                sem_idx[par]).wait()

    def fire_rows(par):
        for j in range(GATHERS):
            pltpu.async_copy(
                wout_hbm.at[cidx_v.at[par, pl.ds(j * GLEN, GLEN)]],
                rows_v.at[par, pl.ds(j * GLEN, GLEN)],
                sem_row[par])

    def drain_rows(par):
        for j in range(GATHERS):
            pltpu.make_async_copy(
                wout_hbm.at[pl.ds(0, GLEN)],
                rows_v.at[par, pl.ds(j * GLEN, GLEN)],
                sem_row[par]).wait()

    fire_idx(0, 0)
    drain_idx(0)
    fire_rows(0)
    fire_idx(1, 1)

    def pair_body(cc, carry):
        for par in (0, 1):
            chunk = cc * 2 + par
            npar = 1 - par

            drain_rows(par)

            @pl.when(chunk + 1 < N_CHUNKS)
            def _fire():
                drain_idx(npar)
                fire_rows(npar)

            @pl.when(chunk + 2 < N_CHUNKS)
            def _pref():
                fire_idx(chunk + 2, par)

            @pl.when(chunk >= 2)
            def _drain_sc():
                pltpu.make_async_copy(
                    out_hbm.at[pl.ds(0, CHUNK_ROWS)], sc_v.at[par],
                    sem_sc[par]).wait()

            for b_loc in range(CHUNK_B):
                b_idx = chunk * CHUNK_B + b_loc
                wv = [inrows_v[b_idx, pl.ds(q * LANES, LANES)]
                      for q in range(DIM // LANES)]

                def group_body(g, gc, b_loc=b_loc, wv=wv):
                    base_row = (b_loc * GROUPS_B + g) * LANES
                    res = jnp.zeros((LANES,), jnp.float32)
                    for r in range(LANES):
                        row = base_row + r
                        p = ((rows_v[par, row, pl.ds(0, LANES)] * wv[0]
                              + rows_v[par, row, pl.ds(LANES, LANES)] * wv[1])
                             + (rows_v[par, row, pl.ds(2 * LANES, LANES)]
                                * wv[2]
                                + rows_v[par, row, pl.ds(3 * LANES, LANES)]
                                * wv[3]))
                        s = jnp.sum(p)
                        res = jnp.where(iota == r, s, res)
                    sc_v[par, pl.ds(base_row, LANES)] = res
                    return gc

                lax.fori_loop(0, GROUPS_B, group_body, 0)

            gchunk = wid * N_CHUNKS + chunk
            pltpu.async_copy(
                sc_v.at[par],
                out_hbm.at[pl.ds(gchunk * CHUNK_ROWS, CHUNK_ROWS)],
                sem_sc[par])
        return carry

    lax.fori_loop(0, N_CHUNKS // 2, pair_body, 0)
    for par in (0, 1):
        pltpu.make_async_copy(
            out_hbm.at[pl.ds(0, CHUNK_ROWS)], sc_v.at[par],
            sem_sc[par]).wait()


def _sc_scores(inword, owp, negword, W_in, W_out):
    mesh = plsc.VectorSubcoreMesh(core_axis_name="c", subcore_axis_name="s")
    k = functools.partial(
        pl.kernel,
        mesh=mesh,
        out_type=jax.ShapeDtypeStruct((TOTAL_ROWS,), jnp.float32),
        compiler_params=pltpu.CompilerParams(
            needs_layout_passes=False, use_tc_tiling_on_sc=False),
        scratch_types=[
            pltpu.VMEM((B_PER_W,), jnp.int32),
            pltpu.VMEM((B_PER_W, DIM), jnp.float32),
            pltpu.VMEM((2, CHUNK_ROWS), jnp.int32),
            pltpu.VMEM((2, CHUNK_ROWS, DIM), jnp.float32),
            pltpu.VMEM((2, CHUNK_ROWS), jnp.float32),
            pltpu.SemaphoreType.DMA,
            pltpu.SemaphoreType.DMA,
            pltpu.SemaphoreType.DMA,
            pltpu.SemaphoreType.DMA,
            pltpu.SemaphoreType.DMA,
            pltpu.SemaphoreType.DMA,
        ],
    )(_sc_body)
    return k(inword, owp, negword, W_in, W_out)


def _tc_loss_body(s_ref, o_ref):
    pid = pl.program_id(0)
    x = s_ref[...]
    r = lax.broadcasted_iota(jnp.int32, (TC_BLK, TC_COLS), 0)
    c = lax.broadcasted_iota(jnp.int32, (TC_BLK, TC_COLS), 1)
    flat = (pid * TC_BLK + r) * TC_COLS + c
    j = flat % ROWS_B
    z = jnp.where(j < NPOS, x, -x)
    ls = jnp.minimum(z, 0.0) - jnp.log(1.0 + jnp.exp(-jnp.abs(z)))
    pad = jnp.logical_and(j >= NPOS, j < NPOSP)
    val = jnp.where(pad, 0.0, ls)

    @pl.when(pid == 0)
    def _init():
        o_ref[0, 0] = 0.0

    o_ref[0, 0] += jnp.sum(val)

    @pl.when(pid == TC_GRID - 1)
    def _fini():
        o_ref[0, 0] = o_ref[0, 0] * (-1.0 / (BATCH * CTX))


def kernel(inword, outword, negword, W_in, W_out):
    owp = jnp.concatenate(
        [outword, jnp.zeros((BATCH, NPOSP - NPOS), jnp.int32)], axis=1)
    scores = _sc_scores(inword, owp, negword, W_in, W_out)
    loss2d = pl.pallas_call(
        _tc_loss_body,
        grid=(TC_GRID,),
        in_specs=[pl.BlockSpec((TC_BLK, TC_COLS), lambda i: (i, 0))],
        out_specs=pl.BlockSpec(
            (1, 1), lambda i: (0, 0), memory_space=pltpu.SMEM),
        out_shape=jax.ShapeDtypeStruct((1, 1), jnp.float32),
    )(scores.reshape(TC_ROWS, TC_COLS))
    return loss2d[0, 0]
